# Initial kernel scaffold; baseline (speedup 1.0000x reference)
import jax
import jax.numpy as jnp
from jax.experimental import pallas as pl


def kernel(x, edge_index, batch, W1, att_src1, att_dst1, b1, W2, att_src2, att_dst2, b2, fc1_w, fc1_b, fc2_w, fc2_b):
    raise NotImplementedError("write your pallas kernel here")


# speedup vs baseline: 66.9180x; 66.9180x over previous
import functools

import jax
import jax.numpy as jnp
from jax import lax
from jax.experimental import pallas as pl
from jax.experimental.pallas import tpu as pltpu
from jax.experimental.pallas import tpu_sc as plsc

N = 10000
E = 320000
HEADS = 8
CH = 16
HC = HEADS * CH
BATCHES = 16

NW = 32
BK = 128
NB = 81
EPAD = NW * NB * BK
NP = 10112
RPT = NP // 16


def _prolog_body(x_ref, w_ref, a_ref, xw_ref, asd_ref):
    xw = jnp.dot(x_ref[...], w_ref[...], preferred_element_type=jnp.float32)
    xw_ref[...] = xw
    asd_ref[...] = jnp.dot(xw, a_ref[...], preferred_element_type=jnp.float32)


def _prolog2_body(o0_ref, o1_ref, b_ref, w_ref, a_ref, xw_ref, asd_ref):
    t = o0_ref[...] + o1_ref[...] + b_ref[...]
    h = jnp.where(t > 0, t, jnp.exp(t) - 1.0)
    xw = jnp.dot(h, w_ref[...], preferred_element_type=jnp.float32)
    xw_ref[...] = xw
    asd_ref[...] = jnp.dot(xw, a_ref[...], preferred_element_type=jnp.float32)


def _rcomb_body(d_ref, o_ref):
    d = d_ref[...]
    o_ref[...] = 1.0 / (d[0] + d[1] + 1e-16)


def _tail_body(o0_ref, o1_ref, b_ref, bt_ref, w1_ref, b1_ref, w2_ref, b2_ref,
               out_ref):
    t = o0_ref[...] + o1_ref[...] + b_ref[...]
    h = jnp.where(t > 0, t, jnp.exp(t) - 1.0)
    bt = bt_ref[...]
    neg = jnp.float32(-jnp.inf)
    means = []
    maxes = []
    for g in range(BATCHES):
        m = bt == g
        s = jnp.sum(jnp.where(m, h, 0.0), axis=0)
        cnt = jnp.sum(jnp.where(m, 1.0, 0.0), axis=0)
        mx = jnp.max(jnp.where(m, h, neg), axis=0)
        means.append(s / (cnt + 1e-16))
        maxes.append(mx)
    gmean = jnp.stack(means)
    gmax = jnp.stack(maxes)
    gcat = jnp.concatenate([gmean, gmax], axis=1)
    g1 = jnp.dot(gcat, w1_ref[...], preferred_element_type=jnp.float32)
    g1 = jnp.maximum(g1 + b1_ref[...], 0.0)
    lg = jnp.dot(g1, w2_ref[...], preferred_element_type=jnp.float32)
    lg = lg + b2_ref[...]
    col = lax.broadcasted_iota(jnp.int32, lg.shape, 1)
    lgm = jnp.where(col < 2, lg, neg)
    mx = jnp.max(lgm, axis=1, keepdims=True)
    lse = jnp.log(jnp.sum(jnp.exp(lgm - mx), axis=1, keepdims=True))
    out_ref[...] = lgm - mx - lse


_MESH = plsc.VectorSubcoreMesh(core_axis_name="c", subcore_axis_name="s")

_GDN = lax.GatherDimensionNumbers(
    offset_dims=(), collapsed_slice_dims=(0,), start_index_map=(0,))


def _bcast_lane(v, h):
    idx = jnp.full((16, 1), h, jnp.int32)
    return lax.gather(v, idx, _GDN, (1,),
                      mode=lax.GatherScatterMode.PROMISE_IN_BOUNDS)


def _sc_pass1(src3, dst3, ats, atd, z16, ex_out, dpart,
              sidx, didx, as_v, ad_v, ex_v, den_sh, sem1, sem2):
    c = lax.axis_index("c")
    s = lax.axis_index("s")
    wid = c * 16 + s
    r0 = s * RPT
    pltpu.sync_copy(z16.at[pl.ds(r0, RPT)], den_sh.at[pl.ds(r0, RPT)])
    plsc.subcore_barrier()
    pltpu.sync_copy(src3.at[wid], sidx)
    pltpu.sync_copy(dst3.at[wid], didx)

    def blk(b, carry):
        si = sidx.at[b]
        di = didx.at[b]
        cp1 = pltpu.async_copy(ats.at[si], as_v, sem1)
        cp2 = pltpu.async_copy(atd.at[di], ad_v, sem2)
        cp1.wait()
        cp2.wait()

        def ci(i, cc2):
            e = as_v[i] + ad_v[i]
            e = jnp.maximum(e, 0.2 * e)
            ex_v[i] = jnp.exp(e)
            return cc2

        lax.fori_loop(0, BK, ci, 0)
        pltpu.sync_copy(ex_v, ex_out.at[wid, b])
        pltpu.sync_copy(ex_v, den_sh.at[di], add=True)
        return carry

    lax.fori_loop(0, NB, blk, 0)
    plsc.subcore_barrier()
    pltpu.sync_copy(den_sh.at[pl.ds(r0, RPT)], dpart.at[c, pl.ds(r0, RPT)])


def _sc_pass2(src3, dst3, xw_tab, ex_in, rden, z128, opart,
              sidx, didx, ex_v, rd_v, xw_v, acc_sh, sem1, sem2):
    c = lax.axis_index("c")
    s = lax.axis_index("s")
    wid = c * 16 + s
    r0 = s * RPT
    pltpu.sync_copy(z128.at[pl.ds(r0, RPT)], acc_sh.at[pl.ds(r0, RPT)])
    plsc.subcore_barrier()
    pltpu.sync_copy(src3.at[wid], sidx)
    pltpu.sync_copy(dst3.at[wid], didx)

    def blk(b, carry):
        si = sidx.at[b]
        di = didx.at[b]
        cpx = pltpu.async_copy(xw_tab.at[si], xw_v, sem1)
        cpr = pltpu.async_copy(rden.at[di], rd_v, sem2)
        pltpu.sync_copy(ex_in.at[wid, b], ex_v)
        cpr.wait()
        cpx.wait()

        def ci(i, cc2):
            aw = ex_v[i] * rd_v[i]
            for hh in range(HEADS):
                bc = _bcast_lane(aw, hh)
                seg = xw_v[i, pl.ds(hh * 16, 16)]
                xw_v[i, pl.ds(hh * 16, 16)] = seg * bc
            return cc2

        lax.fori_loop(0, BK, ci, 0)
        pltpu.sync_copy(xw_v, acc_sh.at[di], add=True)
        return carry

    lax.fori_loop(0, NB, blk, 0)
    plsc.subcore_barrier()
    pltpu.sync_copy(acc_sh.at[pl.ds(r0, RPT)], opart.at[c, pl.ds(r0, RPT)])


_SC_PARAMS = pltpu.CompilerParams(use_tc_tiling_on_sc=False)

_pass1_call = pl.kernel(
    _sc_pass1,
    out_type=(
        jax.ShapeDtypeStruct((NW, NB, BK, 16), jnp.float32),
        jax.ShapeDtypeStruct((2, NP, 16), jnp.float32),
    ),
    mesh=_MESH,
    compiler_params=_SC_PARAMS,
    scratch_types=[
        pltpu.VMEM((NB, BK), jnp.int32),
        pltpu.VMEM((NB, BK), jnp.int32),
        pltpu.VMEM((BK, 16), jnp.float32),
        pltpu.VMEM((BK, 16), jnp.float32),
        pltpu.VMEM((BK, 16), jnp.float32),
        pltpu.VMEM_SHARED((NP, 16), jnp.float32),
        pltpu.SemaphoreType.DMA,
        pltpu.SemaphoreType.DMA,
    ],
)

_pass2_call = pl.kernel(
    _sc_pass2,
    out_type=jax.ShapeDtypeStruct((2, NP, HC), jnp.float32),
    mesh=_MESH,
    compiler_params=_SC_PARAMS,
    scratch_types=[
        pltpu.VMEM((NB, BK), jnp.int32),
        pltpu.VMEM((NB, BK), jnp.int32),
        pltpu.VMEM((BK, 16), jnp.float32),
        pltpu.VMEM((BK, 16), jnp.float32),
        pltpu.VMEM((BK, HC), jnp.float32),
        pltpu.VMEM_SHARED((NP, HC), jnp.float32),
        pltpu.SemaphoreType.DMA,
        pltpu.SemaphoreType.DMA,
    ],
)

_BLK = 512
_GRID = (NP + _BLK - 1) // _BLK

_prolog_call = pl.pallas_call(
    _prolog_body,
    grid=(_GRID,),
    in_specs=[
        pl.BlockSpec((_BLK, HC), lambda i: (i, 0)),
        pl.BlockSpec((HC, HC), lambda i: (0, 0)),
        pl.BlockSpec((HC, 2 * HEADS), lambda i: (0, 0)),
    ],
    out_specs=[
        pl.BlockSpec((_BLK, HC), lambda i: (i, 0)),
        pl.BlockSpec((_BLK, 2 * HEADS), lambda i: (i, 0)),
    ],
    out_shape=[
        jax.ShapeDtypeStruct((NP, HC), jnp.float32),
        jax.ShapeDtypeStruct((NP, 2 * HEADS), jnp.float32),
    ],
)

_prolog2_call = pl.pallas_call(
    _prolog2_body,
    grid=(_GRID,),
    in_specs=[
        pl.BlockSpec((_BLK, HC), lambda i: (i, 0)),
        pl.BlockSpec((_BLK, HC), lambda i: (i, 0)),
        pl.BlockSpec((1, HC), lambda i: (0, 0)),
        pl.BlockSpec((HC, HC), lambda i: (0, 0)),
        pl.BlockSpec((HC, 2 * HEADS), lambda i: (0, 0)),
    ],
    out_specs=[
        pl.BlockSpec((_BLK, HC), lambda i: (i, 0)),
        pl.BlockSpec((_BLK, 2 * HEADS), lambda i: (i, 0)),
    ],
    out_shape=[
        jax.ShapeDtypeStruct((NP, HC), jnp.float32),
        jax.ShapeDtypeStruct((NP, 2 * HEADS), jnp.float32),
    ],
)

_rcomb_call = pl.pallas_call(
    _rcomb_body,
    grid=(_GRID,),
    in_specs=[pl.BlockSpec((2, _BLK, 16), lambda i: (0, i, 0))],
    out_specs=pl.BlockSpec((_BLK, 16), lambda i: (i, 0)),
    out_shape=jax.ShapeDtypeStruct((NP, 16), jnp.float32),
)

_tail_call = pl.pallas_call(
    _tail_body,
    out_shape=jax.ShapeDtypeStruct((BATCHES, HC), jnp.float32),
)


def _gat_layer_sc(xw, asd, src3, dst3, z16, z128):
    zc = jnp.zeros((NP, HEADS), jnp.float32)
    ats = jnp.concatenate([asd[:, :HEADS], zc], axis=1)
    atd = jnp.concatenate([asd[:, HEADS:], zc], axis=1)
    ex_buf, dpart = _pass1_call(src3, dst3, ats, atd, z16)
    rden = _rcomb_call(dpart)
    opart = _pass2_call(src3, dst3, xw, ex_buf, rden, z128)
    return opart


def _att_mat(a_src, a_dst):
    rows = jnp.arange(HC)
    acat = jnp.zeros((HC, 2 * HEADS), jnp.float32)
    acat = acat.at[rows, rows // CH].set(a_src.reshape(HC))
    acat = acat.at[rows, HEADS + rows // CH].set(a_dst.reshape(HC))
    return acat


def kernel(x, edge_index, batch, W1, att_src1, att_dst1, b1, W2, att_src2,
           att_dst2, b2, fc1_w, fc1_b, fc2_w, fc2_b):
    loops = jnp.arange(N, dtype=jnp.int32)
    src = jnp.concatenate([edge_index[0], loops])
    dst = jnp.concatenate([edge_index[1], loops])
    padn = jnp.full((EPAD - (E + N),), N, jnp.int32)
    src3 = jnp.concatenate([src, padn]).reshape(NW, NB, BK)
    dst3 = jnp.concatenate([dst, padn]).reshape(NW, NB, BK)
    xpad = jnp.zeros((NP, HC), jnp.float32).at[:N].set(x)
    z16 = jnp.zeros((NP, 16), jnp.float32)
    z128 = jnp.zeros((NP, HC), jnp.float32)
    acat1 = _att_mat(att_src1, att_dst1)
    acat2 = _att_mat(att_src2, att_dst2)
    btpad = jnp.full((NP, 1), BATCHES, jnp.int32).at[:N, 0].set(batch)
    fc1_wp = jnp.zeros((2 * HC, HC), jnp.float32).at[:, :100].set(fc1_w)
    fc1_bp = jnp.zeros((1, HC), jnp.float32).at[0, :100].set(fc1_b)
    fc2_wp = jnp.zeros((HC, HC), jnp.float32).at[:100, :2].set(fc2_w)
    fc2_bp = jnp.zeros((1, HC), jnp.float32).at[0, :2].set(fc2_b)

    xw1, asd1 = _prolog_call(xpad, W1, acat1)
    op1 = _gat_layer_sc(xw1, asd1, src3, dst3, z16, z128)

    xw2, asd2 = _prolog2_call(op1[0], op1[1], b1.reshape(1, HC), W2, acat2)
    op2 = _gat_layer_sc(xw2, asd2, src3, dst3, z16, z128)

    out = _tail_call(op2[0], op2[1], b2.reshape(1, HC), btpad,
                     fc1_wp, fc1_bp, fc2_wp, fc2_bp)
    return out[:, :2]


# speedup vs baseline: 67.1742x; 1.0038x over previous
import functools

import jax
import jax.numpy as jnp
from jax import lax
from jax.experimental import pallas as pl
from jax.experimental.pallas import tpu as pltpu
from jax.experimental.pallas import tpu_sc as plsc

N = 10000
E = 320000
HEADS = 8
CH = 16
HC = HEADS * CH
BATCHES = 16

NW = 32
BK = 128
NB = 82
EPAD = NW * NB * BK
NP = 10112
RPT = NP // 16


def _prolog_body(x_ref, w_ref, a_ref, xw_ref, asd_ref):
    xw = jnp.dot(x_ref[...], w_ref[...], preferred_element_type=jnp.float32)
    xw_ref[...] = xw
    asd_ref[...] = jnp.dot(xw, a_ref[...], preferred_element_type=jnp.float32)


def _prolog2_body(o0_ref, o1_ref, b_ref, w_ref, a_ref, xw_ref, asd_ref):
    t = o0_ref[...] + o1_ref[...] + b_ref[...]
    h = jnp.where(t > 0, t, jnp.exp(t) - 1.0)
    xw = jnp.dot(h, w_ref[...], preferred_element_type=jnp.float32)
    xw_ref[...] = xw
    asd_ref[...] = jnp.dot(xw, a_ref[...], preferred_element_type=jnp.float32)


def _rcomb_body(d_ref, o_ref):
    d = d_ref[...]
    o_ref[...] = 1.0 / (d[0] + d[1] + 1e-16)


def _tail_body(o0_ref, o1_ref, b_ref, bt_ref, w1_ref, b1_ref, w2_ref, b2_ref,
               out_ref):
    t = o0_ref[...] + o1_ref[...] + b_ref[...]
    h = jnp.where(t > 0, t, jnp.exp(t) - 1.0)
    bt = bt_ref[...]
    neg = jnp.float32(-jnp.inf)
    means = []
    maxes = []
    for g in range(BATCHES):
        m = bt == g
        s = jnp.sum(jnp.where(m, h, 0.0), axis=0)
        cnt = jnp.sum(jnp.where(m, 1.0, 0.0), axis=0)
        mx = jnp.max(jnp.where(m, h, neg), axis=0)
        means.append(s / (cnt + 1e-16))
        maxes.append(mx)
    gmean = jnp.stack(means)
    gmax = jnp.stack(maxes)
    gcat = jnp.concatenate([gmean, gmax], axis=1)
    g1 = jnp.dot(gcat, w1_ref[...], preferred_element_type=jnp.float32)
    g1 = jnp.maximum(g1 + b1_ref[...], 0.0)
    lg = jnp.dot(g1, w2_ref[...], preferred_element_type=jnp.float32)
    lg = lg + b2_ref[...]
    col = lax.broadcasted_iota(jnp.int32, lg.shape, 1)
    lgm = jnp.where(col < 2, lg, neg)
    mx = jnp.max(lgm, axis=1, keepdims=True)
    lse = jnp.log(jnp.sum(jnp.exp(lgm - mx), axis=1, keepdims=True))
    out_ref[...] = lgm - mx - lse


_MESH = plsc.VectorSubcoreMesh(core_axis_name="c", subcore_axis_name="s")

_GDN = lax.GatherDimensionNumbers(
    offset_dims=(), collapsed_slice_dims=(0,), start_index_map=(0,))


def _bcast_lane(v, h):
    idx = jnp.full((16, 1), h, jnp.int32)
    return lax.gather(v, idx, _GDN, (1,),
                      mode=lax.GatherScatterMode.PROMISE_IN_BOUNDS)


def _sc_pass1(src3, dst3, ats, atd, z16, ex_out, dpart,
              sidx, didx, as_v0, ad_v0, as_v1, ad_v1, ex_v, den_sh,
              sem10, sem20, sem11, sem21):
    c = lax.axis_index("c")
    s = lax.axis_index("s")
    wid = c * 16 + s
    r0 = s * RPT
    pltpu.sync_copy(z16.at[pl.ds(r0, RPT)], den_sh.at[pl.ds(r0, RPT)])
    plsc.subcore_barrier()
    pltpu.sync_copy(src3.at[wid], sidx)
    pltpu.sync_copy(dst3.at[wid], didx)

    def issue(b, as_v, ad_v, s1, s2):
        pltpu.async_copy(ats.at[sidx.at[b]], as_v, s1)
        pltpu.async_copy(atd.at[didx.at[b]], ad_v, s2)

    def run(b, as_v, ad_v, s1, s2):
        pltpu.make_async_copy(ats.at[sidx.at[b]], as_v, s1).wait()
        pltpu.make_async_copy(atd.at[didx.at[b]], ad_v, s2).wait()

        def ci(i, cc2):
            e = as_v[i] + ad_v[i]
            e = jnp.maximum(e, 0.2 * e)
            ex_v[i] = jnp.exp(e)
            return cc2

        lax.fori_loop(0, BK, ci, 0)
        pltpu.sync_copy(ex_v, ex_out.at[wid, b])
        pltpu.sync_copy(ex_v, den_sh.at[didx.at[b]], add=True)

    issue(0, as_v0, ad_v0, sem10, sem20)

    def blk2(j, carry):
        b0 = 2 * j
        issue(b0 + 1, as_v1, ad_v1, sem11, sem21)
        run(b0, as_v0, ad_v0, sem10, sem20)

        @pl.when(b0 + 2 < NB)
        def _():
            issue(b0 + 2, as_v0, ad_v0, sem10, sem20)

        run(b0 + 1, as_v1, ad_v1, sem11, sem21)
        return carry

    lax.fori_loop(0, NB // 2, blk2, 0)
    plsc.subcore_barrier()
    pltpu.sync_copy(den_sh.at[pl.ds(r0, RPT)], dpart.at[c, pl.ds(r0, RPT)])


def _sc_pass2(pidx3, xw_tab, ex_in, rden, z128, opart,
              pidx_v, sb0, db0, sb1, db1, ex_v, rd_v0, xw_v0, rd_v1, xw_v1,
              acc_sh, sx0, sr0, sx1, sr1):
    c = lax.axis_index("c")
    s = lax.axis_index("s")
    wid = c * 16 + s
    r0 = s * RPT
    pltpu.sync_copy(z128.at[pl.ds(r0, RPT)], acc_sh.at[pl.ds(r0, RPT)])
    plsc.subcore_barrier()
    pltpu.sync_copy(pidx3.at[wid], pidx_v)

    def issue(b, sb, db, rd_v, xw_v, sx, sr):
        for k in range(HEADS):
            v = pidx_v[b, pl.ds(k * 16, 16)]
            sb[0, pl.ds(k * 16, 16)] = lax.bitwise_and(v, 16383)
            db[0, pl.ds(k * 16, 16)] = lax.shift_right_logical(v, 14)
        pltpu.async_copy(xw_tab.at[sb.at[0]], xw_v, sx)
        pltpu.async_copy(rden.at[db.at[0]], rd_v, sr)

    def run(b, sb, db, rd_v, xw_v, sx, sr):
        pltpu.sync_copy(ex_in.at[wid, b], ex_v)
        pltpu.make_async_copy(rden.at[db.at[0]], rd_v, sr).wait()
        pltpu.make_async_copy(xw_tab.at[sb.at[0]], xw_v, sx).wait()

        def ci(i, cc2):
            aw = ex_v[i] * rd_v[i]
            for hh in range(HEADS):
                bc = _bcast_lane(aw, hh)
                seg = xw_v[i, pl.ds(hh * 16, 16)]
                xw_v[i, pl.ds(hh * 16, 16)] = seg * bc
            return cc2

        lax.fori_loop(0, BK, ci, 0)
        pltpu.sync_copy(xw_v, acc_sh.at[db.at[0]], add=True)

    issue(0, sb0, db0, rd_v0, xw_v0, sx0, sr0)

    def blk2(j, carry):
        b0 = 2 * j
        issue(b0 + 1, sb1, db1, rd_v1, xw_v1, sx1, sr1)
        run(b0, sb0, db0, rd_v0, xw_v0, sx0, sr0)

        @pl.when(b0 + 2 < NB)
        def _():
            issue(b0 + 2, sb0, db0, rd_v0, xw_v0, sx0, sr0)

        run(b0 + 1, sb1, db1, rd_v1, xw_v1, sx1, sr1)
        return carry

    lax.fori_loop(0, NB // 2, blk2, 0)
    plsc.subcore_barrier()
    pltpu.sync_copy(acc_sh.at[pl.ds(r0, RPT)], opart.at[c, pl.ds(r0, RPT)])


_SC_PARAMS = pltpu.CompilerParams(use_tc_tiling_on_sc=False)

_pass1_call = pl.kernel(
    _sc_pass1,
    out_type=(
        jax.ShapeDtypeStruct((NW, NB, BK, 16), jnp.float32),
        jax.ShapeDtypeStruct((2, NP, 16), jnp.float32),
    ),
    mesh=_MESH,
    compiler_params=_SC_PARAMS,
    scratch_types=[
        pltpu.VMEM((NB, BK), jnp.int32),
        pltpu.VMEM((NB, BK), jnp.int32),
        pltpu.VMEM((BK, 16), jnp.float32),
        pltpu.VMEM((BK, 16), jnp.float32),
        pltpu.VMEM((BK, 16), jnp.float32),
        pltpu.VMEM((BK, 16), jnp.float32),
        pltpu.VMEM((BK, 16), jnp.float32),
        pltpu.VMEM_SHARED((NP, 16), jnp.float32),
        pltpu.SemaphoreType.DMA,
        pltpu.SemaphoreType.DMA,
        pltpu.SemaphoreType.DMA,
        pltpu.SemaphoreType.DMA,
    ],
)

_pass2_call = pl.kernel(
    _sc_pass2,
    out_type=jax.ShapeDtypeStruct((2, NP, HC), jnp.float32),
    mesh=_MESH,
    compiler_params=_SC_PARAMS,
    scratch_types=[
        pltpu.VMEM((NB, BK), jnp.int32),
        pltpu.VMEM((1, BK), jnp.int32),
        pltpu.VMEM((1, BK), jnp.int32),
        pltpu.VMEM((1, BK), jnp.int32),
        pltpu.VMEM((1, BK), jnp.int32),
        pltpu.VMEM((BK, 16), jnp.float32),
        pltpu.VMEM((BK, 16), jnp.float32),
        pltpu.VMEM((BK, HC), jnp.float32),
        pltpu.VMEM((BK, 16), jnp.float32),
        pltpu.VMEM((BK, HC), jnp.float32),
        pltpu.VMEM_SHARED((NP, HC), jnp.float32),
        pltpu.SemaphoreType.DMA,
        pltpu.SemaphoreType.DMA,
        pltpu.SemaphoreType.DMA,
        pltpu.SemaphoreType.DMA,
    ],
)

_BLK = 512
_GRID = (NP + _BLK - 1) // _BLK

_prolog_call = pl.pallas_call(
    _prolog_body,
    grid=(_GRID,),
    in_specs=[
        pl.BlockSpec((_BLK, HC), lambda i: (i, 0)),
        pl.BlockSpec((HC, HC), lambda i: (0, 0)),
        pl.BlockSpec((HC, 2 * HEADS), lambda i: (0, 0)),
    ],
    out_specs=[
        pl.BlockSpec((_BLK, HC), lambda i: (i, 0)),
        pl.BlockSpec((_BLK, 2 * HEADS), lambda i: (i, 0)),
    ],
    out_shape=[
        jax.ShapeDtypeStruct((NP, HC), jnp.float32),
        jax.ShapeDtypeStruct((NP, 2 * HEADS), jnp.float32),
    ],
)

_prolog2_call = pl.pallas_call(
    _prolog2_body,
    grid=(_GRID,),
    in_specs=[
        pl.BlockSpec((_BLK, HC), lambda i: (i, 0)),
        pl.BlockSpec((_BLK, HC), lambda i: (i, 0)),
        pl.BlockSpec((1, HC), lambda i: (0, 0)),
        pl.BlockSpec((HC, HC), lambda i: (0, 0)),
        pl.BlockSpec((HC, 2 * HEADS), lambda i: (0, 0)),
    ],
    out_specs=[
        pl.BlockSpec((_BLK, HC), lambda i: (i, 0)),
        pl.BlockSpec((_BLK, 2 * HEADS), lambda i: (i, 0)),
    ],
    out_shape=[
        jax.ShapeDtypeStruct((NP, HC), jnp.float32),
        jax.ShapeDtypeStruct((NP, 2 * HEADS), jnp.float32),
    ],
)

_rcomb_call = pl.pallas_call(
    _rcomb_body,
    grid=(_GRID,),
    in_specs=[pl.BlockSpec((2, _BLK, 16), lambda i: (0, i, 0))],
    out_specs=pl.BlockSpec((_BLK, 16), lambda i: (i, 0)),
    out_shape=jax.ShapeDtypeStruct((NP, 16), jnp.float32),
)

_tail_call = pl.pallas_call(
    _tail_body,
    out_shape=jax.ShapeDtypeStruct((BATCHES, HC), jnp.float32),
)


def _gat_layer_sc(xw, asd, src3, dst3, pidx3, z16, z128):
    zc = jnp.zeros((NP, HEADS), jnp.float32)
    ats = jnp.concatenate([asd[:, :HEADS], zc], axis=1)
    atd = jnp.concatenate([asd[:, HEADS:], zc], axis=1)
    ex_buf, dpart = _pass1_call(src3, dst3, ats, atd, z16)
    rden = _rcomb_call(dpart)
    opart = _pass2_call(pidx3, xw, ex_buf, rden, z128)
    return opart


def _att_mat(a_src, a_dst):
    rows = jnp.arange(HC)
    acat = jnp.zeros((HC, 2 * HEADS), jnp.float32)
    acat = acat.at[rows, rows // CH].set(a_src.reshape(HC))
    acat = acat.at[rows, HEADS + rows // CH].set(a_dst.reshape(HC))
    return acat


def kernel(x, edge_index, batch, W1, att_src1, att_dst1, b1, W2, att_src2,
           att_dst2, b2, fc1_w, fc1_b, fc2_w, fc2_b):
    loops = jnp.arange(N, dtype=jnp.int32)
    src = jnp.concatenate([edge_index[0], loops])
    dst = jnp.concatenate([edge_index[1], loops])
    padn = jnp.full((EPAD - (E + N),), N, jnp.int32)
    src3 = jnp.concatenate([src, padn]).reshape(NW, NB, BK)
    dst3 = jnp.concatenate([dst, padn]).reshape(NW, NB, BK)
    pidx3 = src3 + dst3 * 16384
    xpad = jnp.zeros((NP, HC), jnp.float32).at[:N].set(x)
    z16 = jnp.zeros((NP, 16), jnp.float32)
    z128 = jnp.zeros((NP, HC), jnp.float32)
    acat1 = _att_mat(att_src1, att_dst1)
    acat2 = _att_mat(att_src2, att_dst2)
    btpad = jnp.full((NP, 1), BATCHES, jnp.int32).at[:N, 0].set(batch)
    fc1_wp = jnp.zeros((2 * HC, HC), jnp.float32).at[:, :100].set(fc1_w)
    fc1_bp = jnp.zeros((1, HC), jnp.float32).at[0, :100].set(fc1_b)
    fc2_wp = jnp.zeros((HC, HC), jnp.float32).at[:100, :2].set(fc2_w)
    fc2_bp = jnp.zeros((1, HC), jnp.float32).at[0, :2].set(fc2_b)

    xw1, asd1 = _prolog_call(xpad, W1, acat1)
    op1 = _gat_layer_sc(xw1, asd1, src3, dst3, pidx3, z16, z128)

    xw2, asd2 = _prolog2_call(op1[0], op1[1], b1.reshape(1, HC), W2, acat2)
    op2 = _gat_layer_sc(xw2, asd2, src3, dst3, pidx3, z16, z128)

    out = _tail_call(op2[0], op2[1], b2.reshape(1, HC), btpad,
                     fc1_wp, fc1_bp, fc2_wp, fc2_bp)
    return out[:, :2]


# speedup vs baseline: 70.0856x; 1.0433x over previous
import functools

import jax
import jax.numpy as jnp
from jax import lax
from jax.experimental import pallas as pl
from jax.experimental.pallas import tpu as pltpu
from jax.experimental.pallas import tpu_sc as plsc

N = 10000
E = 320000
HEADS = 8
CH = 16
HC = HEADS * CH
BATCHES = 16

NW = 32
BK = 128
NB = 82
EPAD = NW * NB * BK
NP = 10112
RPT = NP // 16


def _prolog_body(x_ref, w_ref, a_ref, xw_ref, asd_ref):
    xw = jnp.dot(x_ref[...], w_ref[...], preferred_element_type=jnp.float32)
    xw_ref[...] = xw
    asd_ref[...] = jnp.dot(xw, a_ref[...], preferred_element_type=jnp.float32)


def _prolog2_body(o0_ref, o1_ref, b_ref, w_ref, a_ref, xw_ref, asd_ref):
    t = o0_ref[...] + o1_ref[...] + b_ref[...]
    h = jnp.where(t > 0, t, jnp.exp(t) - 1.0)
    xw = jnp.dot(h, w_ref[...], preferred_element_type=jnp.float32)
    xw_ref[...] = xw
    asd_ref[...] = jnp.dot(xw, a_ref[...], preferred_element_type=jnp.float32)


def _rcomb_body(d_ref, o_ref):
    d = d_ref[...]
    o_ref[...] = 1.0 / (d[0] + d[1] + 1e-16)


def _tail_body(o0_ref, o1_ref, b_ref, bt_ref, w1_ref, b1_ref, w2_ref, b2_ref,
               out_ref):
    t = o0_ref[...] + o1_ref[...] + b_ref[...]
    h = jnp.where(t > 0, t, jnp.exp(t) - 1.0)
    bt = bt_ref[...]
    neg = jnp.float32(-jnp.inf)
    means = []
    maxes = []
    for g in range(BATCHES):
        m = bt == g
        s = jnp.sum(jnp.where(m, h, 0.0), axis=0)
        cnt = jnp.sum(jnp.where(m, 1.0, 0.0), axis=0)
        mx = jnp.max(jnp.where(m, h, neg), axis=0)
        means.append(s / (cnt + 1e-16))
        maxes.append(mx)
    gmean = jnp.stack(means)
    gmax = jnp.stack(maxes)
    gcat = jnp.concatenate([gmean, gmax], axis=1)
    g1 = jnp.dot(gcat, w1_ref[...], preferred_element_type=jnp.float32)
    g1 = jnp.maximum(g1 + b1_ref[...], 0.0)
    lg = jnp.dot(g1, w2_ref[...], preferred_element_type=jnp.float32)
    lg = lg + b2_ref[...]
    col = lax.broadcasted_iota(jnp.int32, lg.shape, 1)
    lgm = jnp.where(col < 2, lg, neg)
    mx = jnp.max(lgm, axis=1, keepdims=True)
    lse = jnp.log(jnp.sum(jnp.exp(lgm - mx), axis=1, keepdims=True))
    out_ref[...] = lgm - mx - lse


_MESH = plsc.VectorSubcoreMesh(core_axis_name="c", subcore_axis_name="s")

_GDN = lax.GatherDimensionNumbers(
    offset_dims=(), collapsed_slice_dims=(0,), start_index_map=(0,))


def _bcast_lane(v, h):
    idx = jnp.full((16, 1), h, jnp.int32)
    return lax.gather(v, idx, _GDN, (1,),
                      mode=lax.GatherScatterMode.PROMISE_IN_BOUNDS)


def _sc_pass1(src3, dst3, ats, atd, z16, ex_out, dpart,
              sidx, didx, as_v0, ad_v0, as_v1, ad_v1, ex_v, den_sh,
              sem10, sem20, sem11, sem21):
    c = lax.axis_index("c")
    s = lax.axis_index("s")
    wid = c * 16 + s
    r0 = s * RPT
    pltpu.sync_copy(z16.at[pl.ds(r0, RPT)], den_sh.at[pl.ds(r0, RPT)])
    plsc.subcore_barrier()
    pltpu.sync_copy(src3.at[wid], sidx)
    pltpu.sync_copy(dst3.at[wid], didx)

    def issue(b, as_v, ad_v, s1, s2):
        pltpu.async_copy(ats.at[sidx.at[b]], as_v, s1)
        pltpu.async_copy(atd.at[didx.at[b]], ad_v, s2)

    def run(b, as_v, ad_v, s1, s2):
        pltpu.make_async_copy(ats.at[sidx.at[b]], as_v, s1).wait()
        pltpu.make_async_copy(atd.at[didx.at[b]], ad_v, s2).wait()

        @plsc.parallel_loop(0, BK, unroll=4)
        def _(i):
            e = as_v[i] + ad_v[i]
            e = jnp.maximum(e, 0.2 * e)
            ex_v[i] = jnp.exp(e)

        pltpu.sync_copy(ex_v, ex_out.at[wid, b])
        pltpu.sync_copy(ex_v, den_sh.at[didx.at[b]], add=True)

    issue(0, as_v0, ad_v0, sem10, sem20)

    def blk2(j, carry):
        b0 = 2 * j
        issue(b0 + 1, as_v1, ad_v1, sem11, sem21)
        run(b0, as_v0, ad_v0, sem10, sem20)

        @pl.when(b0 + 2 < NB)
        def _():
            issue(b0 + 2, as_v0, ad_v0, sem10, sem20)

        run(b0 + 1, as_v1, ad_v1, sem11, sem21)
        return carry

    lax.fori_loop(0, NB // 2, blk2, 0)
    plsc.subcore_barrier()
    pltpu.sync_copy(den_sh.at[pl.ds(r0, RPT)], dpart.at[c, pl.ds(r0, RPT)])


def _sc_pass2(pidx3, xw_tab, ex_in, rden, z128, opart,
              pidx_v, sb0, db0, sb1, db1, ex_v, rd_v0, xw_v0, rd_v1, xw_v1,
              acc_sh, sx0, sr0, sx1, sr1):
    c = lax.axis_index("c")
    s = lax.axis_index("s")
    wid = c * 16 + s
    r0 = s * RPT
    pltpu.sync_copy(z128.at[pl.ds(r0, RPT)], acc_sh.at[pl.ds(r0, RPT)])
    plsc.subcore_barrier()
    pltpu.sync_copy(pidx3.at[wid], pidx_v)

    def issue(b, sb, db, rd_v, xw_v, sx, sr):
        for k in range(HEADS):
            v = pidx_v[b, pl.ds(k * 16, 16)]
            sb[0, pl.ds(k * 16, 16)] = lax.bitwise_and(v, 16383)
            db[0, pl.ds(k * 16, 16)] = lax.shift_right_logical(v, 14)
        pltpu.async_copy(xw_tab.at[sb.at[0]], xw_v, sx)
        pltpu.async_copy(rden.at[db.at[0]], rd_v, sr)

    def run(b, sb, db, rd_v, xw_v, sx, sr):
        pltpu.sync_copy(ex_in.at[wid, b], ex_v)
        pltpu.make_async_copy(rden.at[db.at[0]], rd_v, sr).wait()
        pltpu.make_async_copy(xw_tab.at[sb.at[0]], xw_v, sx).wait()

        @plsc.parallel_loop(0, BK, unroll=4)
        def _(i):
            aw = ex_v[i] * rd_v[i]
            for hh in range(HEADS):
                bc = _bcast_lane(aw, hh)
                seg = xw_v[i, pl.ds(hh * 16, 16)]
                xw_v[i, pl.ds(hh * 16, 16)] = seg * bc

        pltpu.sync_copy(xw_v, acc_sh.at[db.at[0]], add=True)

    issue(0, sb0, db0, rd_v0, xw_v0, sx0, sr0)

    def blk2(j, carry):
        b0 = 2 * j
        issue(b0 + 1, sb1, db1, rd_v1, xw_v1, sx1, sr1)
        run(b0, sb0, db0, rd_v0, xw_v0, sx0, sr0)

        @pl.when(b0 + 2 < NB)
        def _():
            issue(b0 + 2, sb0, db0, rd_v0, xw_v0, sx0, sr0)

        run(b0 + 1, sb1, db1, rd_v1, xw_v1, sx1, sr1)
        return carry

    lax.fori_loop(0, NB // 2, blk2, 0)
    plsc.subcore_barrier()
    pltpu.sync_copy(acc_sh.at[pl.ds(r0, RPT)], opart.at[c, pl.ds(r0, RPT)])


_SC_PARAMS = pltpu.CompilerParams(use_tc_tiling_on_sc=False)

_pass1_call = pl.kernel(
    _sc_pass1,
    out_type=(
        jax.ShapeDtypeStruct((NW, NB, BK, 16), jnp.float32),
        jax.ShapeDtypeStruct((2, NP, 16), jnp.float32),
    ),
    mesh=_MESH,
    compiler_params=_SC_PARAMS,
    scratch_types=[
        pltpu.VMEM((NB, BK), jnp.int32),
        pltpu.VMEM((NB, BK), jnp.int32),
        pltpu.VMEM((BK, 16), jnp.float32),
        pltpu.VMEM((BK, 16), jnp.float32),
        pltpu.VMEM((BK, 16), jnp.float32),
        pltpu.VMEM((BK, 16), jnp.float32),
        pltpu.VMEM((BK, 16), jnp.float32),
        pltpu.VMEM_SHARED((NP, 16), jnp.float32),
        pltpu.SemaphoreType.DMA,
        pltpu.SemaphoreType.DMA,
        pltpu.SemaphoreType.DMA,
        pltpu.SemaphoreType.DMA,
    ],
)

_pass2_call = pl.kernel(
    _sc_pass2,
    out_type=jax.ShapeDtypeStruct((2, NP, HC), jnp.float32),
    mesh=_MESH,
    compiler_params=_SC_PARAMS,
    scratch_types=[
        pltpu.VMEM((NB, BK), jnp.int32),
        pltpu.VMEM((1, BK), jnp.int32),
        pltpu.VMEM((1, BK), jnp.int32),
        pltpu.VMEM((1, BK), jnp.int32),
        pltpu.VMEM((1, BK), jnp.int32),
        pltpu.VMEM((BK, 16), jnp.float32),
        pltpu.VMEM((BK, 16), jnp.float32),
        pltpu.VMEM((BK, HC), jnp.float32),
        pltpu.VMEM((BK, 16), jnp.float32),
        pltpu.VMEM((BK, HC), jnp.float32),
        pltpu.VMEM_SHARED((NP, HC), jnp.float32),
        pltpu.SemaphoreType.DMA,
        pltpu.SemaphoreType.DMA,
        pltpu.SemaphoreType.DMA,
        pltpu.SemaphoreType.DMA,
    ],
)

_BLK = 512
_GRID = (NP + _BLK - 1) // _BLK

_prolog_call = pl.pallas_call(
    _prolog_body,
    grid=(_GRID,),
    in_specs=[
        pl.BlockSpec((_BLK, HC), lambda i: (i, 0)),
        pl.BlockSpec((HC, HC), lambda i: (0, 0)),
        pl.BlockSpec((HC, 2 * HEADS), lambda i: (0, 0)),
    ],
    out_specs=[
        pl.BlockSpec((_BLK, HC), lambda i: (i, 0)),
        pl.BlockSpec((_BLK, 2 * HEADS), lambda i: (i, 0)),
    ],
    out_shape=[
        jax.ShapeDtypeStruct((NP, HC), jnp.float32),
        jax.ShapeDtypeStruct((NP, 2 * HEADS), jnp.float32),
    ],
)

_prolog2_call = pl.pallas_call(
    _prolog2_body,
    grid=(_GRID,),
    in_specs=[
        pl.BlockSpec((_BLK, HC), lambda i: (i, 0)),
        pl.BlockSpec((_BLK, HC), lambda i: (i, 0)),
        pl.BlockSpec((1, HC), lambda i: (0, 0)),
        pl.BlockSpec((HC, HC), lambda i: (0, 0)),
        pl.BlockSpec((HC, 2 * HEADS), lambda i: (0, 0)),
    ],
    out_specs=[
        pl.BlockSpec((_BLK, HC), lambda i: (i, 0)),
        pl.BlockSpec((_BLK, 2 * HEADS), lambda i: (i, 0)),
    ],
    out_shape=[
        jax.ShapeDtypeStruct((NP, HC), jnp.float32),
        jax.ShapeDtypeStruct((NP, 2 * HEADS), jnp.float32),
    ],
)

_rcomb_call = pl.pallas_call(
    _rcomb_body,
    grid=(_GRID,),
    in_specs=[pl.BlockSpec((2, _BLK, 16), lambda i: (0, i, 0))],
    out_specs=pl.BlockSpec((_BLK, 16), lambda i: (i, 0)),
    out_shape=jax.ShapeDtypeStruct((NP, 16), jnp.float32),
)

_tail_call = pl.pallas_call(
    _tail_body,
    out_shape=jax.ShapeDtypeStruct((BATCHES, HC), jnp.float32),
)


def _gat_layer_sc(xw, asd, src3, dst3, pidx3, z16, z128):
    zc = jnp.zeros((NP, HEADS), jnp.float32)
    ats = jnp.concatenate([asd[:, :HEADS], zc], axis=1)
    atd = jnp.concatenate([asd[:, HEADS:], zc], axis=1)
    ex_buf, dpart = _pass1_call(src3, dst3, ats, atd, z16)
    rden = _rcomb_call(dpart)
    opart = _pass2_call(pidx3, xw, ex_buf, rden, z128)
    return opart


def _att_mat(a_src, a_dst):
    rows = jnp.arange(HC)
    acat = jnp.zeros((HC, 2 * HEADS), jnp.float32)
    acat = acat.at[rows, rows // CH].set(a_src.reshape(HC))
    acat = acat.at[rows, HEADS + rows // CH].set(a_dst.reshape(HC))
    return acat


def kernel(x, edge_index, batch, W1, att_src1, att_dst1, b1, W2, att_src2,
           att_dst2, b2, fc1_w, fc1_b, fc2_w, fc2_b):
    loops = jnp.arange(N, dtype=jnp.int32)
    src = jnp.concatenate([edge_index[0], loops])
    dst = jnp.concatenate([edge_index[1], loops])
    padn = jnp.full((EPAD - (E + N),), N, jnp.int32)
    src3 = jnp.concatenate([src, padn]).reshape(NW, NB, BK)
    dst3 = jnp.concatenate([dst, padn]).reshape(NW, NB, BK)
    pidx3 = src3 + dst3 * 16384
    xpad = jnp.zeros((NP, HC), jnp.float32).at[:N].set(x)
    z16 = jnp.zeros((NP, 16), jnp.float32)
    z128 = jnp.zeros((NP, HC), jnp.float32)
    acat1 = _att_mat(att_src1, att_dst1)
    acat2 = _att_mat(att_src2, att_dst2)
    btpad = jnp.full((NP, 1), BATCHES, jnp.int32).at[:N, 0].set(batch)
    fc1_wp = jnp.zeros((2 * HC, HC), jnp.float32).at[:, :100].set(fc1_w)
    fc1_bp = jnp.zeros((1, HC), jnp.float32).at[0, :100].set(fc1_b)
    fc2_wp = jnp.zeros((HC, HC), jnp.float32).at[:100, :2].set(fc2_w)
    fc2_bp = jnp.zeros((1, HC), jnp.float32).at[0, :2].set(fc2_b)

    xw1, asd1 = _prolog_call(xpad, W1, acat1)
    op1 = _gat_layer_sc(xw1, asd1, src3, dst3, pidx3, z16, z128)

    xw2, asd2 = _prolog2_call(op1[0], op1[1], b1.reshape(1, HC), W2, acat2)
    op2 = _gat_layer_sc(xw2, asd2, src3, dst3, pidx3, z16, z128)

    out = _tail_call(op2[0], op2[1], b2.reshape(1, HC), btpad,
                     fc1_wp, fc1_bp, fc2_wp, fc2_bp)
    return out[:, :2]
